# baseline (speedup 1.0000x reference)
import functools

import jax
import jax.numpy as jnp
from jax import lax
from jax.experimental import pallas as pl
from jax.experimental.pallas import tpu as pltpu
from jax.experimental.pallas import tpu_sc as plsc

POOLED = 7
RED = 16.0
NEG = float(jnp.finfo(jnp.float32).min)

NCHUNK = 16
ROWPAD = 816
RPC = 8
NBUF = 2


def _roi_meta(rois, image_ids, H, W):
    R4 = rois.reshape(-1, 4)
    scale = jnp.float32(1.0 / RED)
    xs = jnp.round(R4[:, 0] * scale).astype(jnp.int32)
    ys = jnp.round(R4[:, 1] * scale).astype(jnp.int32)
    xe = jnp.round(R4[:, 2] * scale).astype(jnp.int32)
    ye = jnp.round(R4[:, 3] * scale).astype(jnp.int32)
    roi_w = jnp.maximum(xe - xs + 1, 1).astype(jnp.float32)
    roi_h = jnp.maximum(ye - ys + 1, 1).astype(jnp.float32)
    bin_h = roi_h / POOLED
    bin_w = roi_w / POOLED
    p = jnp.arange(POOLED, dtype=jnp.float32)
    hs = jnp.clip(jnp.floor(p[None, :] * bin_h[:, None]).astype(jnp.int32) + ys[:, None], 0, H)
    he = jnp.clip(jnp.ceil((p[None, :] + 1.0) * bin_h[:, None]).astype(jnp.int32) + ys[:, None], 0, H)
    ws = jnp.clip(jnp.floor(p[None, :] * bin_w[:, None]).astype(jnp.int32) + xs[:, None], 0, W)
    we = jnp.clip(jnp.ceil((p[None, :] + 1.0) * bin_w[:, None]).astype(jnp.int32) + xs[:, None], 0, W)
    hl = he - hs
    wl = we - ws
    nroi = R4.shape[0]
    nimg = image_ids.shape[0]
    per = nroi // nimg
    bimg = jnp.repeat(image_ids.astype(jnp.int32), per)
    pad = jnp.zeros((nroi, 1), jnp.int32)
    meta = jnp.concatenate(
        [hs, pad, hl, pad, ws, pad, wl, bimg[:, None]], axis=1
    )
    return meta


def _sc_body(feat_hbm, meta_hbm, out_hbm, tab_v, meta_v, row_v, outb_v, sem0, sem1):
    cc = lax.axis_index("s")
    g = lax.axis_index("c")
    sems = (sem0, sem1)

    pltpu.sync_copy(feat_hbm.at[cc, pl.ds(g * 80000, 80000)], tab_v)
    pltpu.sync_copy(meta_hbm.at[pl.ds(g * 16384, 16384)], meta_v)

    zero16 = jnp.zeros((16,), jnp.float32)
    neg16 = jnp.full((16,), NEG, jnp.float32)

    def do_roi(r, buf):
        rbase = r * 32
        mrow0 = meta_v[pl.ds(rbase, 16)]
        mrow1 = meta_v[pl.ds(rbase + 16, 16)]
        li = mrow1[15] - 2 * g
        ws0 = mrow1[0]
        wend = mrow1[6] + mrow1[14]
        nw = wend - ws0

        for ph in range(POOLED):
            hs = mrow0[ph]
            hl = mrow0[8 + ph]

            rb0 = (li * 2500 + hs * 50 + ws0) * 16

            @pl.when(hl > 0)
            def _(rb0=rb0, hl=hl, ph=ph):
                def cp(j, _):
                    row_v[ph, pl.ds(j * 16, 16)] = tab_v[pl.ds(rb0 + j * 16, 16)]
                    return 0

                lax.fori_loop(0, nw, cp, 0)

                def mx(k, _):
                    rbk = rb0 + k * 800

                    def mxj(j, _):
                        v = tab_v[pl.ds(rbk + j * 16, 16)]
                        row_v[ph, pl.ds(j * 16, 16)] = jnp.maximum(
                            row_v[ph, pl.ds(j * 16, 16)], v
                        )
                        return 0

                    lax.fori_loop(0, nw, mxj, 0)
                    return 0

                lax.fori_loop(1, hl, mx, 0)

            @pl.when(hl == 0)
            def _(ph=ph):
                def zf(j, _):
                    row_v[ph, pl.ds(j * 16, 16)] = zero16
                    return 0

                lax.fori_loop(0, nw, zf, 0)

        for pw in range(POOLED):
            ws = mrow1[pw]
            wl = mrow1[8 + pw]
            rel = (ws - ws0) * 16
            floor = jnp.where(wl == 0, jnp.float32(0.0), jnp.float32(NEG))
            floor16 = jnp.broadcast_to(floor, (16,))
            for ph in range(POOLED):
                def wk(k, a, rel=rel, ph=ph):
                    return jnp.maximum(a, row_v[ph, pl.ds(rel + k * 16, 16)])

                acc = lax.fori_loop(0, wl, wk, neg16)
                res = jnp.maximum(acc, floor16)
                outb_v[pl.ds(buf * 896 + (ph * POOLED + pw) * 16, 16)] = res

    def pair(r2, _):
        for b in range(NBUF):
            r = r2 * NBUF + b

            @pl.when(r2 >= 1)
            def _(b=b):
                rgp = g * 512 + (r2 - 1) * NBUF + b
                pltpu.make_async_copy(
                    outb_v.at[pl.ds(b * 896, 896)], out_hbm.at[rgp, cc], sems[b]
                ).wait()

            do_roi(r, b)
            rg = g * 512 + r
            pltpu.async_copy(
                outb_v.at[pl.ds(b * 896, 896)], out_hbm.at[rg, cc], sems[b]
            )
        return 0

    lax.fori_loop(0, 512 // NBUF, pair, 0)
    for b in range(NBUF):
        rgp = g * 512 + 512 - NBUF + b
        pltpu.make_async_copy(
            outb_v.at[pl.ds(b * 896, 896)], out_hbm.at[rgp, cc], sems[b]
        ).wait()


@jax.jit
def _sc_run(featW, meta_flat):
    kfn = functools.partial(
        pl.kernel,
        out_type=jax.ShapeDtypeStruct((1024, NCHUNK, 896), jnp.float32),
        mesh=plsc.VectorSubcoreMesh(core_axis_name="c", subcore_axis_name="s"),
        scratch_types=[
            pltpu.VMEM((80000,), jnp.float32),
            pltpu.VMEM((16384,), jnp.int32),
            pltpu.VMEM((POOLED, ROWPAD), jnp.float32),
            pltpu.VMEM((NBUF * 896,), jnp.float32),
            pltpu.SemaphoreType.DMA,
            pltpu.SemaphoreType.DMA,
        ],
    )(_sc_body)
    return kfn(featW, meta_flat)


def kernel(image, image_ids, rois):
    B, C, H, W = image.shape
    featW = (
        image.transpose(1, 0, 2, 3)
        .reshape(NCHUNK, 16, B, H, W)
        .transpose(0, 2, 3, 4, 1)
        .reshape(NCHUNK, B * H * W * 16)
    )
    meta = _roi_meta(rois, image_ids, H, W)
    out = _sc_run(featW, meta.reshape(-1))
    nroi = meta.shape[0]
    out = out[:, :, : POOLED * POOLED * 16]
    out = out.reshape(nroi, NCHUNK, POOLED * POOLED, 16).transpose(0, 1, 3, 2)
    return out.reshape(nroi, C, POOLED, POOLED)


# speedup vs baseline: 1.0457x; 1.0457x over previous
import functools

import jax
import jax.numpy as jnp
from jax import lax
from jax.experimental import pallas as pl
from jax.experimental.pallas import tpu as pltpu
from jax.experimental.pallas import tpu_sc as plsc

POOLED = 7
RED = 16.0
NEG = float(jnp.finfo(jnp.float32).min)

NCHUNK = 16
ROWPAD = 880
NEGOFF = 848
NBUF = 2
KMAX = 10


def _roi_meta(rois, image_ids, H, W):
    R4 = rois.reshape(-1, 4)
    scale = jnp.float32(1.0 / RED)
    xs = jnp.round(R4[:, 0] * scale).astype(jnp.int32)
    ys = jnp.round(R4[:, 1] * scale).astype(jnp.int32)
    xe = jnp.round(R4[:, 2] * scale).astype(jnp.int32)
    ye = jnp.round(R4[:, 3] * scale).astype(jnp.int32)
    roi_w = jnp.maximum(xe - xs + 1, 1).astype(jnp.float32)
    roi_h = jnp.maximum(ye - ys + 1, 1).astype(jnp.float32)
    bin_h = roi_h / POOLED
    bin_w = roi_w / POOLED
    p = jnp.arange(POOLED, dtype=jnp.float32)
    hs = jnp.clip(jnp.floor(p[None, :] * bin_h[:, None]).astype(jnp.int32) + ys[:, None], 0, H)
    he = jnp.clip(jnp.ceil((p[None, :] + 1.0) * bin_h[:, None]).astype(jnp.int32) + ys[:, None], 0, H)
    ws = jnp.clip(jnp.floor(p[None, :] * bin_w[:, None]).astype(jnp.int32) + xs[:, None], 0, W)
    we = jnp.clip(jnp.ceil((p[None, :] + 1.0) * bin_w[:, None]).astype(jnp.int32) + xs[:, None], 0, W)
    hl = he - hs
    wl = we - ws
    nroi = R4.shape[0]
    nimg = image_ids.shape[0]
    per = nroi // nimg
    bimg = jnp.repeat(image_ids.astype(jnp.int32), per)
    pad = jnp.zeros((nroi, 1), jnp.int32)
    meta = jnp.concatenate(
        [hs, pad, hl, pad, ws, pad, wl, bimg[:, None]], axis=1
    )
    return meta


def _sc_body(feat_hbm, meta_hbm, out_hbm, tab_v, meta_v, row_v, outb_v, sem0, sem1):
    cc = lax.axis_index("s")
    g = lax.axis_index("c")
    sems = (sem0, sem1)

    pltpu.sync_copy(feat_hbm.at[cc, pl.ds(g * 80000, 80000)], tab_v.at[pl.ds(0, 80000)])
    pltpu.sync_copy(meta_hbm.at[pl.ds(g * 16384, 16384)], meta_v)

    zero16 = jnp.zeros((16,), jnp.float32)
    neg16 = jnp.full((16,), NEG, jnp.float32)
    for ph in range(POOLED):
        row_v[ph, pl.ds(NEGOFF, 16)] = neg16

    def do_roi(r, buf):
        rbase = r * 32
        mrow0 = meta_v[pl.ds(rbase, 16)]
        mrow1 = meta_v[pl.ds(rbase + 16, 16)]
        li = mrow1[15] - 2 * g
        ws0 = mrow1[0]
        wend = mrow1[6] + mrow1[14]
        nw = wend - ws0

        nw4 = (nw + 3) // 4
        for ph in range(POOLED):
            hs = mrow0[ph]
            hl = mrow0[8 + ph]

            rb0 = (li * 2500 + hs * 50 + ws0) * 16

            @pl.when(hl > 0)
            def _(rb0=rb0, hl=hl, ph=ph):
                def jblk(j4, _):
                    base = rb0 + j4 * 64
                    a = tuple(
                        tab_v[pl.ds(base + t * 16, 16)] for t in range(4)
                    )

                    def kstep(k, accs, base=base):
                        bk = base + k * 800
                        return tuple(
                            jnp.maximum(accs[t], tab_v[pl.ds(bk + t * 16, 16)])
                            for t in range(4)
                        )

                    a = lax.fori_loop(1, hl, kstep, a)
                    for t in range(4):
                        row_v[ph, pl.ds(j4 * 64 + t * 16, 16)] = a[t]
                    return 0

                lax.fori_loop(0, nw4, jblk, 0)

            @pl.when(hl == 0)
            def _(ph=ph):
                def zf(j4, _):
                    for t in range(4):
                        row_v[ph, pl.ds(j4 * 64 + t * 16, 16)] = zero16
                    return 0

                lax.fori_loop(0, nw4, zf, 0)

        for pw in range(POOLED):
            ws = mrow1[pw]
            wl = mrow1[8 + pw]
            rel = jnp.where(wl == 0, jnp.int32(NEGOFF), (ws - ws0) * 16)
            floor = jnp.where(wl == 0, jnp.float32(0.0), jnp.float32(NEG))
            floor16 = jnp.broadcast_to(floor, (16,))
            wlm = jnp.maximum(wl - 1, 0)
            offs = [rel + jnp.minimum(jnp.int32(k), wlm) * 16 for k in range(KMAX)]
            for ph in range(POOLED):
                acc = row_v[ph, pl.ds(offs[0], 16)]
                for k in range(1, KMAX):
                    acc = jnp.maximum(acc, row_v[ph, pl.ds(offs[k], 16)])
                res = jnp.maximum(acc, floor16)
                outb_v[pl.ds(buf * 896 + (ph * POOLED + pw) * 16, 16)] = res

    def pair(r2, _):
        for b in range(NBUF):
            r = r2 * NBUF + b

            @pl.when(r2 >= 1)
            def _(b=b):
                rgp = g * 512 + (r2 - 1) * NBUF + b
                pltpu.make_async_copy(
                    outb_v.at[pl.ds(b * 896, 896)], out_hbm.at[rgp, cc], sems[b]
                ).wait()

            do_roi(r, b)
            rg = g * 512 + r
            pltpu.async_copy(
                outb_v.at[pl.ds(b * 896, 896)], out_hbm.at[rg, cc], sems[b]
            )
        return 0

    lax.fori_loop(0, 512 // NBUF, pair, 0)
    for b in range(NBUF):
        rgp = g * 512 + 512 - NBUF + b
        pltpu.make_async_copy(
            outb_v.at[pl.ds(b * 896, 896)], out_hbm.at[rgp, cc], sems[b]
        ).wait()


@jax.jit
def _sc_run(featW, meta_flat):
    kfn = functools.partial(
        pl.kernel,
        out_type=jax.ShapeDtypeStruct((1024, NCHUNK, 896), jnp.float32),
        mesh=plsc.VectorSubcoreMesh(core_axis_name="c", subcore_axis_name="s"),
        scratch_types=[
            pltpu.VMEM((80896,), jnp.float32),
            pltpu.VMEM((16384,), jnp.int32),
            pltpu.VMEM((POOLED, ROWPAD), jnp.float32),
            pltpu.VMEM((NBUF * 896,), jnp.float32),
            pltpu.SemaphoreType.DMA,
            pltpu.SemaphoreType.DMA,
        ],
    )(_sc_body)
    return kfn(featW, meta_flat)


def kernel(image, image_ids, rois):
    B, C, H, W = image.shape
    featW = (
        image.transpose(1, 0, 2, 3)
        .reshape(NCHUNK, 16, B, H, W)
        .transpose(0, 2, 3, 4, 1)
        .reshape(NCHUNK, B * H * W * 16)
    )
    meta = _roi_meta(rois, image_ids, H, W)
    out = _sc_run(featW, meta.reshape(-1))
    nroi = meta.shape[0]
    out = out[:, :, : POOLED * POOLED * 16]
    out = out.reshape(nroi, NCHUNK, POOLED * POOLED, 16).transpose(0, 1, 3, 2)
    return out.reshape(nroi, C, POOLED, POOLED)
